# baseline (speedup 1.0000x reference)
import functools

import jax
import jax.numpy as jnp
from jax.experimental import pallas as pl
from jax.experimental.pallas import tpu as pltpu


_GRP = 16


def _body(outs, tgts, out_ref, soxr, soyr, btxr, btyr, o2r, t2r,
          acc1_ref, acc2_ref):
    bs = outs.shape[0]
    n = outs.shape[1] // 2

    def bf(x):
        return x.astype(jnp.bfloat16).astype(jnp.float32)

    ox = outs[:, :n].T
    oy = outs[:, n:].T
    tx = tgts[:, :n].T
    ty = tgts[:, n:].T
    o2r[...] = ox * ox + oy * oy
    t2r[...] = tx * tx + ty * ty
    soxr[...] = -2.0 * bf(ox)
    soyr[...] = -2.0 * bf(oy)
    btxr[...] = bf(tx)
    btyr[...] = bf(ty)
    sox = soxr[...]
    soy = soyr[...]
    btx = btxr[...]
    bty = btyr[...]
    o2 = o2r[...]
    t2 = t2r[...]
    acc1_ref[...] = jnp.full(acc1_ref.shape, 1e30, jnp.float32)
    acc2_ref[...] = jnp.full(acc2_ref.shape, 1e30, jnp.float32)

    def grp(g, _):
        sl = pl.ds(g * _GRP, _GRP)
        btxg = btxr[sl, :]
        btyg = btyr[sl, :]
        t2g = t2r[sl, :]
        soxg = soxr[sl, :]
        soyg = soyr[sl, :]
        o2g = o2r[sl, :]
        m1a = acc1_ref[...]
        m2a = acc2_ref[...]
        m1b = None
        m2b = None
        for k in range(_GRP):
            u1 = sox * btxg[k:k + 1, :] + t2g[k:k + 1, :]
            u1 = soy * btyg[k:k + 1, :] + u1
            u2 = btx * soxg[k:k + 1, :] + o2g[k:k + 1, :]
            u2 = bty * soyg[k:k + 1, :] + u2
            if k % 2 == 0:
                m1a = jnp.minimum(m1a, u1)
                m2a = jnp.minimum(m2a, u2)
            else:
                m1b = u1 if m1b is None else jnp.minimum(m1b, u1)
                m2b = u2 if m2b is None else jnp.minimum(m2b, u2)
        acc1_ref[...] = jnp.minimum(m1a, m1b)
        acc2_ref[...] = jnp.minimum(m2a, m2b)
        return 0

    jax.lax.fori_loop(0, n // _GRP, grp, 0)
    d2_ot = jnp.maximum(acc1_ref[...] + o2, 0.0)
    d2_to = jnp.maximum(acc2_ref[...] + t2, 0.0)
    out_ref[0, 0] = (jnp.sum(jnp.sqrt(d2_ot + 1e-12))
                     + jnp.sum(jnp.sqrt(d2_to + 1e-12)))


@functools.partial(jax.jit, static_argnames=())
def kernel(outputs, targets):
    bs, f = outputs.shape
    n = f // 2

    total = pl.pallas_call(
        _body,
        out_shape=jax.ShapeDtypeStruct((1, 1), jnp.float32),
        in_specs=[pl.BlockSpec((bs, f), lambda: (0, 0))] * 2,
        out_specs=pl.BlockSpec(memory_space=pltpu.SMEM),
        scratch_shapes=[pltpu.VMEM((n, bs), jnp.float32)] * 8,
    )(outputs, targets)

    return total[0, 0] * (0.5 / (bs * n))


# speedup vs baseline: 1.0095x; 1.0095x over previous
import functools

import jax
import jax.numpy as jnp
from jax.experimental import pallas as pl
from jax.experimental.pallas import tpu as pltpu


_GRP = 32


def _body(outs, tgts, out_ref, soxr, soyr, btxr, btyr, o2r, t2r,
          acc1_ref, acc2_ref):
    bs = outs.shape[0]
    n = outs.shape[1] // 2

    def bf(x):
        return x.astype(jnp.bfloat16).astype(jnp.float32)

    ox = outs[:, :n].T
    oy = outs[:, n:].T
    tx = tgts[:, :n].T
    ty = tgts[:, n:].T
    o2r[...] = ox * ox + oy * oy
    t2r[...] = tx * tx + ty * ty
    soxr[...] = -2.0 * bf(ox)
    soyr[...] = -2.0 * bf(oy)
    btxr[...] = bf(tx)
    btyr[...] = bf(ty)
    sox = soxr[...]
    soy = soyr[...]
    btx = btxr[...]
    bty = btyr[...]
    o2 = o2r[...]
    t2 = t2r[...]
    acc1_ref[...] = jnp.full(acc1_ref.shape, 1e30, jnp.float32)
    acc2_ref[...] = jnp.full(acc2_ref.shape, 1e30, jnp.float32)

    def grp(g, _):
        sl = pl.ds(g * _GRP, _GRP)
        btxg = btxr[sl, :]
        btyg = btyr[sl, :]
        t2g = t2r[sl, :]
        soxg = soxr[sl, :]
        soyg = soyr[sl, :]
        o2g = o2r[sl, :]
        m1a = acc1_ref[...]
        m2a = acc2_ref[...]
        m1b = None
        m2b = None
        for k in range(_GRP):
            u1 = sox * btxg[k:k + 1, :] + t2g[k:k + 1, :]
            u1 = soy * btyg[k:k + 1, :] + u1
            u2 = btx * soxg[k:k + 1, :] + o2g[k:k + 1, :]
            u2 = bty * soyg[k:k + 1, :] + u2
            if k % 2 == 0:
                m1a = jnp.minimum(m1a, u1)
                m2a = jnp.minimum(m2a, u2)
            else:
                m1b = u1 if m1b is None else jnp.minimum(m1b, u1)
                m2b = u2 if m2b is None else jnp.minimum(m2b, u2)
        acc1_ref[...] = jnp.minimum(m1a, m1b)
        acc2_ref[...] = jnp.minimum(m2a, m2b)
        return 0

    jax.lax.fori_loop(0, n // _GRP, grp, 0)
    d2_ot = jnp.maximum(acc1_ref[...] + o2, 0.0)
    d2_to = jnp.maximum(acc2_ref[...] + t2, 0.0)
    out_ref[0, 0] = (jnp.sum(jnp.sqrt(d2_ot + 1e-12))
                     + jnp.sum(jnp.sqrt(d2_to + 1e-12)))


@functools.partial(jax.jit, static_argnames=())
def kernel(outputs, targets):
    bs, f = outputs.shape
    n = f // 2

    total = pl.pallas_call(
        _body,
        out_shape=jax.ShapeDtypeStruct((1, 1), jnp.float32),
        in_specs=[pl.BlockSpec((bs, f), lambda: (0, 0))] * 2,
        out_specs=pl.BlockSpec(memory_space=pltpu.SMEM),
        scratch_shapes=[pltpu.VMEM((n, bs), jnp.float32)] * 8,
    )(outputs, targets)

    return total[0, 0] * (0.5 / (bs * n))


# speedup vs baseline: 1.0172x; 1.0076x over previous
import functools

import jax
import jax.numpy as jnp
from jax.experimental import pallas as pl
from jax.experimental.pallas import tpu as pltpu


_GRP = 32


def _body(outs, tgts, out_ref, soxr, soyr, btxr, btyr, o2r, t2r,
          acc1_ref, acc2_ref):
    bs = outs.shape[0]
    n = outs.shape[1] // 2

    def bf(x):
        return x.astype(jnp.bfloat16).astype(jnp.float32)

    ox = outs[:, :n].T
    oy = outs[:, n:].T
    tx = tgts[:, :n].T
    ty = tgts[:, n:].T
    o2r[...] = ox * ox + oy * oy
    t2r[...] = tx * tx + ty * ty
    soxr[...] = -2.0 * bf(ox)
    soyr[...] = -2.0 * bf(oy)
    btxr[...] = bf(tx)
    btyr[...] = bf(ty)
    sox = soxr[...]
    soy = soyr[...]
    btx = btxr[...]
    bty = btyr[...]
    o2 = o2r[...]
    t2 = t2r[...]
    acc1_ref[...] = jnp.full(acc1_ref.shape, 1e30, jnp.float32)
    acc2_ref[...] = jnp.full(acc2_ref.shape, 1e30, jnp.float32)

    def grp(g, _):
        sl = pl.ds(g * _GRP, _GRP)
        btxg = btxr[sl, :]
        btyg = btyr[sl, :]
        t2g = t2r[sl, :]
        soxg = soxr[sl, :]
        soyg = soyr[sl, :]
        o2g = o2r[sl, :]
        m1 = [acc1_ref[...], None, None, None]
        m2 = [acc2_ref[...], None, None, None]
        for k in range(_GRP):
            u1 = sox * btxg[k:k + 1, :] + t2g[k:k + 1, :]
            u1 = soy * btyg[k:k + 1, :] + u1
            u2 = btx * soxg[k:k + 1, :] + o2g[k:k + 1, :]
            u2 = bty * soyg[k:k + 1, :] + u2
            w = k % 4
            m1[w] = u1 if m1[w] is None else jnp.minimum(m1[w], u1)
            m2[w] = u2 if m2[w] is None else jnp.minimum(m2[w], u2)
        acc1_ref[...] = jnp.minimum(jnp.minimum(m1[0], m1[1]),
                                    jnp.minimum(m1[2], m1[3]))
        acc2_ref[...] = jnp.minimum(jnp.minimum(m2[0], m2[1]),
                                    jnp.minimum(m2[2], m2[3]))
        return 0

    jax.lax.fori_loop(0, n // _GRP, grp, 0)
    d2_ot = jnp.maximum(acc1_ref[...] + o2, 0.0)
    d2_to = jnp.maximum(acc2_ref[...] + t2, 0.0)
    out_ref[0, 0] = (jnp.sum(jnp.sqrt(d2_ot + 1e-12))
                     + jnp.sum(jnp.sqrt(d2_to + 1e-12)))


@functools.partial(jax.jit, static_argnames=())
def kernel(outputs, targets):
    bs, f = outputs.shape
    n = f // 2

    total = pl.pallas_call(
        _body,
        out_shape=jax.ShapeDtypeStruct((1, 1), jnp.float32),
        in_specs=[pl.BlockSpec((bs, f), lambda: (0, 0))] * 2,
        out_specs=pl.BlockSpec(memory_space=pltpu.SMEM),
        scratch_shapes=[pltpu.VMEM((n, bs), jnp.float32)] * 8,
    )(outputs, targets)

    return total[0, 0] * (0.5 / (bs * n))
